# baseline (speedup 1.0000x reference)
import functools

import jax
import jax.numpy as jnp
from jax import lax
from jax.experimental import pallas as pl
from jax.experimental.pallas import tpu as pltpu
from jax.experimental.pallas import tpu_sc as plsc

EMB = 1024
BATCH = 1024
SEQ = 50
N = BATCH * SEQ
NW = 32
PER_W = N // NW
CHUNK = 32
NBUF = 2
NCHUNK = PER_W // CHUNK
NGROUP = NCHUNK // NBUF


def _sc_gather(table, idx):
    mesh = plsc.VectorSubcoreMesh(core_axis_name="c", subcore_axis_name="s")

    @functools.partial(
        pl.kernel,
        mesh=mesh,
        out_type=jax.ShapeDtypeStruct((N, EMB), jnp.float32),
        scratch_types=(
            [pltpu.VMEM((NCHUNK, CHUNK), jnp.int32)]
            + [pltpu.VMEM((CHUNK, EMB), jnp.float32) for _ in range(NBUF)]
            + [pltpu.SemaphoreType.DMA for _ in range(2 * NBUF)]
        ),
    )
    def gather_kernel(table_hbm, idx_hbm, out_hbm, idx_v, *rest):
        bufs = rest[:NBUF]
        gsems = rest[NBUF:2 * NBUF]
        ssems = rest[2 * NBUF:]

        wid = lax.axis_index("s") * 2 + lax.axis_index("c")
        base = wid * PER_W
        pltpu.sync_copy(idx_hbm.at[wid], idx_v)

        def start_gather(c, b):
            pltpu.async_copy(table_hbm.at[idx_v.at[c]], bufs[b], gsems[b])

        def wait_gather(c, b):
            pltpu.make_async_copy(
                table_hbm.at[idx_v.at[c]], bufs[b], gsems[b]).wait()

        def start_scatter(c, b):
            pltpu.async_copy(
                bufs[b], out_hbm.at[pl.ds(base + c * CHUNK, CHUNK)], ssems[b])

        def wait_scatter(c, b):
            pltpu.make_async_copy(
                bufs[b], out_hbm.at[pl.ds(base + c * CHUNK, CHUNK)],
                ssems[b]).wait()

        for k in range(NBUF):
            start_gather(k, k)

        def body(i, carry):
            c0 = NBUF * i
            for k in range(NBUF):
                wait_gather(c0 + k, k)
                start_scatter(c0 + k, k)
            for k in range(NBUF):
                wait_scatter(c0 + k, k)
                start_gather(c0 + NBUF + k, k)
            return carry

        lax.fori_loop(0, NGROUP - 1, body, 0)

        c0 = NCHUNK - NBUF
        for k in range(NBUF):
            wait_gather(c0 + k, k)
            start_scatter(c0 + k, k)
        for k in range(NBUF):
            wait_scatter(c0 + k, k)

    return gather_kernel(table, idx)


def kernel(log_seqs, table):
    idx = log_seqs.astype(jnp.int32).T.reshape(NW, NCHUNK, CHUNK)
    out = _sc_gather(table, idx)
    return out.reshape(SEQ, BATCH, EMB).transpose(1, 0, 2)


# speedup vs baseline: 1.0165x; 1.0165x over previous
import functools

import jax
import jax.numpy as jnp
from jax import lax
from jax.experimental import pallas as pl
from jax.experimental.pallas import tpu as pltpu
from jax.experimental.pallas import tpu_sc as plsc

EMB = 1024
BATCH = 1024
SEQ = 50
N = BATCH * SEQ
NW = 32
PER_W = N // NW
CHUNK = 8
NBUF = 10
NCHUNK = PER_W // CHUNK
NGROUP = NCHUNK // NBUF


def _sc_gather(table, idx):
    mesh = plsc.VectorSubcoreMesh(core_axis_name="c", subcore_axis_name="s")

    @functools.partial(
        pl.kernel,
        mesh=mesh,
        out_type=jax.ShapeDtypeStruct((N, EMB), jnp.float32),
        scratch_types=(
            [pltpu.VMEM((NCHUNK, CHUNK), jnp.int32)]
            + [pltpu.VMEM((CHUNK, EMB), jnp.float32) for _ in range(NBUF)]
            + [pltpu.SemaphoreType.DMA for _ in range(2 * NBUF)]
        ),
    )
    def gather_kernel(table_hbm, idx_hbm, out_hbm, idx_v, *rest):
        bufs = rest[:NBUF]
        gsems = rest[NBUF:2 * NBUF]
        ssems = rest[2 * NBUF:]

        wid = lax.axis_index("s") * 2 + lax.axis_index("c")
        base = wid * PER_W
        pltpu.sync_copy(idx_hbm.at[wid], idx_v)

        def start_gather(c, b):
            pltpu.async_copy(table_hbm.at[idx_v.at[c]], bufs[b], gsems[b])

        def wait_gather(c, b):
            pltpu.make_async_copy(
                table_hbm.at[idx_v.at[c]], bufs[b], gsems[b]).wait()

        def start_scatter(c, b):
            pltpu.async_copy(
                bufs[b], out_hbm.at[pl.ds(base + c * CHUNK, CHUNK)], ssems[b])

        def wait_scatter(c, b):
            pltpu.make_async_copy(
                bufs[b], out_hbm.at[pl.ds(base + c * CHUNK, CHUNK)],
                ssems[b]).wait()

        for k in range(NBUF):
            start_gather(k, k)

        def body(i, carry):
            c0 = NBUF * i
            for k in range(NBUF):
                wait_gather(c0 + k, k)
                start_scatter(c0 + k, k)
            for k in range(NBUF):
                wait_scatter(c0 + k, k)
                start_gather(c0 + NBUF + k, k)
            return carry

        lax.fori_loop(0, NGROUP - 1, body, 0)

        c0 = NCHUNK - NBUF
        for k in range(NBUF):
            wait_gather(c0 + k, k)
            start_scatter(c0 + k, k)
        for k in range(NBUF):
            wait_scatter(c0 + k, k)

    return gather_kernel(table, idx)


def kernel(log_seqs, table):
    idx = log_seqs.astype(jnp.int32).T.reshape(NW, NCHUNK, CHUNK)
    out = _sc_gather(table, idx)
    return out.reshape(SEQ, BATCH, EMB).transpose(1, 0, 2)
